# baseline (speedup 1.0000x reference)
import functools

import jax
import jax.numpy as jnp
from jax import lax
from jax.experimental import pallas as pl
from jax.experimental.pallas import tpu as pltpu
from jax.experimental.pallas import tpu_sc as plsc

NC = 2
NS = 16
LANES = 16
CHUNK = 128


def _sc_mesh():
    return plsc.VectorSubcoreMesh(core_axis_name="c", subcore_axis_name="s")


def _zero_fill(ref, rows, width):
    def body(i, carry):
        for k in range(width // LANES):
            ref[i, pl.ds(k * LANES, LANES)] = jnp.zeros((LANES,), jnp.float32)
        return carry
    lax.fori_loop(0, rows, body, 0)


def _zero_shared_rows(fill_v, nfill, acc_s, base, rpt):
    nfull, rem = rpt // nfill, rpt % nfill
    for k in range(nfull):
        pltpu.sync_copy(fill_v, acc_s.at[pl.ds(base + nfill * k, nfill)])
    if rem:
        pltpu.sync_copy(fill_v.at[pl.ds(0, rem)],
                        acc_s.at[pl.ds(base + nfill * nfull, rem)])


def _make_deg_kernel(np_rows, ch_per_tile, rpt, width=16):
    @functools.partial(
        pl.kernel,
        out_type=jax.ShapeDtypeStruct((NC, np_rows, width), jnp.float32),
        mesh=_sc_mesh(),
        compiler_params=pltpu.CompilerParams(use_tc_tiling_on_sc=False),
        scratch_types=[
            pltpu.VMEM((ch_per_tile, CHUNK), jnp.int32),
            pltpu.VMEM((128, width), jnp.float32),
            pltpu.VMEM((128, width), jnp.float32),
            pltpu.VMEM_SHARED((np_rows, width), jnp.float32),
        ],
    )
    def deg_kernel(dst_hbm, ones_hbm, out_hbm, idx_v, fill_v, ones_v, acc_s):
        c = lax.axis_index("c")
        s = lax.axis_index("s")
        wid = s * NC + c
        base = s * rpt
        _zero_fill(fill_v, 128, width)
        _zero_shared_rows(fill_v, 128, acc_s, base, rpt)
        pltpu.sync_copy(ones_hbm, ones_v)
        plsc.subcore_barrier()
        pltpu.sync_copy(dst_hbm.at[pl.ds(wid * ch_per_tile, ch_per_tile)],
                        idx_v)

        def step(j, carry):
            pltpu.sync_copy(ones_v, acc_s.at[idx_v.at[j]], add=True)
            return carry
        lax.fori_loop(0, ch_per_tile, step, 0)
        plsc.subcore_barrier()
        pltpu.sync_copy(acc_s.at[pl.ds(base, rpt)],
                        out_hbm.at[c].at[pl.ds(base, rpt)])

    return deg_kernel


def _make_agg_kernel(np_rows, nch, rpt, width):
    grp = 2
    assert nch % (2 * grp) == grp, "chunk count sized so the tail is one group"
    assert nch >= 3 * grp

    @functools.partial(
        pl.kernel,
        out_type=jax.ShapeDtypeStruct((NC, np_rows, width), jnp.float32),
        mesh=_sc_mesh(),
        compiler_params=pltpu.CompilerParams(use_tc_tiling_on_sc=False),
        scratch_types=[
            pltpu.VMEM((nch, CHUNK), jnp.int32),
            pltpu.VMEM((nch, CHUNK), jnp.int32),
            pltpu.VMEM((grp * 128, width), jnp.float32),
            pltpu.VMEM((grp * 128, width), jnp.float32),
            pltpu.VMEM_SHARED((np_rows, width), jnp.float32),
        ] + [pltpu.SemaphoreType.DMA] * 4,
    )
    def agg_kernel(src_hbm, dst_hbm, tab_hbm, out_hbm,
                   idxs_v, idxd_v, big_a, big_b, acc_s, ga, gb, sa, sb):
        c = lax.axis_index("c")
        s = lax.axis_index("s")
        tab = tab_hbm.at[c]
        base = s * rpt
        _zero_fill(big_a, grp * 128, width)
        _zero_shared_rows(big_a, grp * 128, acc_s, base, rpt)
        plsc.subcore_barrier()
        pltpu.sync_copy(src_hbm.at[pl.ds(s * nch, nch)], idxs_v)
        pltpu.sync_copy(dst_hbm.at[pl.ds(s * nch, nch)], idxd_v)

        def ggrp(j, big, sem):
            for o in range(grp):
                pltpu.async_copy(tab.at[idxs_v.at[j + o]],
                                 big.at[pl.ds(o * 128, 128)], sem)

        def gwaitgrp(big, sem):
            pltpu.make_async_copy(tab.at[pl.ds(0, grp * 128)], big,
                                  sem).wait()

        def scatgrp(j, big, sem):
            for o in range(grp):
                pltpu.async_copy(big.at[pl.ds(o * 128, 128)],
                                 acc_s.at[idxd_v.at[j + o]], sem, add=True)

        def swaitgrp(big, sem):
            pltpu.make_async_copy(big, acc_s.at[pl.ds(0, grp * 128)],
                                  sem).wait()

        ggrp(0, big_a, ga)

        def body(k, carry):
            j = 2 * grp * k
            ggrp(j + grp, big_b, gb)
            gwaitgrp(big_a, ga)
            scatgrp(j, big_a, sa)
            swaitgrp(big_a, sa)
            ggrp(j + 2 * grp, big_a, ga)
            gwaitgrp(big_b, gb)
            scatgrp(j + grp, big_b, sb)
            swaitgrp(big_b, sb)
            return carry
        lax.fori_loop(0, (nch - grp) // (2 * grp), body, 0)
        gwaitgrp(big_a, ga)
        scatgrp(nch - grp, big_a, sa)
        swaitgrp(big_a, sa)
        plsc.subcore_barrier()
        pltpu.sync_copy(acc_s.at[pl.ds(base, rpt)],
                        out_hbm.at[c].at[pl.ds(base, rpt)])

    return agg_kernel


def _make_tc_prep(q0, q, n):
    def body(ei_ref, src_ref, dst_ref):
        src_ref[:q0] = ei_ref[0]
        dst_ref[:q0] = ei_ref[1]
        if q > q0:
            pad = jnp.full((q - q0, CHUNK), jnp.int32(n))
            src_ref[q0:] = pad
            dst_ref[q0:] = pad
    return body


def _dinv_from_deg(degt_ref):
    deg = degt_ref[0, :, 0:1] + degt_ref[1, :, 0:1] + 1.0
    return lax.rsqrt(deg)


def _make_tc_prescale(n):
    def body(x_ref, w1_ref, degt_ref, xs_ref):
        dinv = _dinv_from_deg(degt_ref)
        xw = jnp.dot(x_ref[...], w1_ref[...],
                     preferred_element_type=jnp.float32)
        xw = xw * dinv[:n]
        h = xs_ref.shape[2]
        np_rows = xs_ref.shape[1]
        for cc in range(NC):
            xs_ref[cc, :n, :] = xw[:, cc * h:(cc + 1) * h]
            xs_ref[cc, n:, :] = jnp.zeros((np_rows - n, h), jnp.float32)
    return body


def _tc_layer2(agg_ref, xs_ref, degt_ref, w2_ref, b1_ref, ys_ref):
    dinv = _dinv_from_deg(degt_ref)
    h = agg_ref.shape[2]
    h1a = jnp.maximum(dinv * (agg_ref[0] + xs_ref[0]) + b1_ref[:, :h], 0.0)
    h1b = jnp.maximum(dinv * (agg_ref[1] + xs_ref[1]) + b1_ref[:, h:], 0.0)
    ys = (jnp.dot(h1a, w2_ref[:h], preferred_element_type=jnp.float32)
          + jnp.dot(h1b, w2_ref[h:],
                    preferred_element_type=jnp.float32)) * dinv
    h2 = ys_ref.shape[2]
    ys_ref[0] = ys[:, :h2]
    ys_ref[1] = ys[:, h2:]


def _make_tc_head(n):
    def body(agg2_ref, ys_ref, degt_ref, batch_ref, solv_ref, b2_ref,
             l1w_ref, l1bias_ref, l2w_ref, l2bias_ref,
             l3w_ref, l3bias_ref, hw1_ref, hb1_ref, hw2_ref, hb2_ref,
             out_ref):
        num_graphs = out_ref.shape[0]
        dinv = _dinv_from_deg(degt_ref)[:n]
        h2 = agg2_ref.shape[2]
        nodes_a = dinv * (agg2_ref[0, :n, :] + ys_ref[0, :n, :])
        nodes_b = dinv * (agg2_ref[1, :n, :] + ys_ref[1, :n, :])
        gids = lax.broadcasted_iota(jnp.int32, (num_graphs, n), 0)
        mask = (batch_ref[...] == gids).astype(jnp.float32)
        cnt = jnp.sum(mask, axis=1, keepdims=True)
        pooled_a = (jnp.dot(mask, nodes_a, preferred_element_type=jnp.float32)
                    + cnt * b2_ref[:, :h2])
        pooled_b = (jnp.dot(mask, nodes_b, preferred_element_type=jnp.float32)
                    + cnt * b2_ref[:, h2:])
        z = (jnp.dot(pooled_a, l1w_ref[:h2],
                     preferred_element_type=jnp.float32)
             + jnp.dot(pooled_b, l1w_ref[h2:2 * h2],
                       preferred_element_type=jnp.float32)
             + jnp.dot(solv_ref[...], l1w_ref[2 * h2:],
                       preferred_element_type=jnp.float32))
        z = jnp.maximum(z + l1bias_ref[...], 0.0)
        z = jnp.maximum(jnp.dot(z, l2w_ref[...],
                                preferred_element_type=jnp.float32)
                        + l2bias_ref[...], 0.0)
        z = jnp.maximum(jnp.dot(z, l3w_ref[...],
                                preferred_element_type=jnp.float32)
                        + l3bias_ref[...], 0.0)
        nh = out_ref.shape[1]
        for e in range(nh):
            hh = jnp.maximum(jnp.dot(z, hw1_ref[e],
                                     preferred_element_type=jnp.float32)
                             + hb1_ref[e:e + 1, :], 0.0)
            out_ref[:, e:e + 1] = (jnp.dot(hh, hw2_ref[e],
                                           preferred_element_type=jnp.float32)
                                   + hb2_ref[e:e + 1, :])
    return body


def kernel(x, edge_index, edge_attr, batch_index, solvent_descriptors,
           mol_fingerprints, W1, b1, W2, b2, lin1_W, lin1_b, lin2_W, lin2_b,
           lin3_W, lin3_b, heads_W1, heads_b1, heads_W2, heads_b2):
    n, din = x.shape
    e = edge_index.shape[1]
    g = solvent_descriptors.shape[0]
    d1 = W1.shape[1]
    d2 = W2.shape[1]
    nh = heads_W1.shape[0]

    rpt = -(-(n + 1) // (NS * 8)) * 8
    np_rows = rpt * NS

    grp = 2
    q0 = -(-e // CHUNK)
    q = q0
    while q % 64 != 32:
        q += 1
    nch = q // NS
    deg_ch = q // (NC * NS)

    if e == q0 * CHUNK:
        ei = edge_index.astype(jnp.int32).reshape(2, q0, CHUNK)
    else:
        ei = jnp.concatenate(
            [edge_index.astype(jnp.int32),
             jnp.full((2, q0 * CHUNK - e), jnp.int32(n))], axis=1
        ).reshape(2, q0, CHUNK)

    srcp, dstp = pl.pallas_call(
        _make_tc_prep(q0, q, n),
        out_shape=[jax.ShapeDtypeStruct((q, CHUNK), jnp.int32)] * 2,
    )(ei)

    ones16 = jnp.ones((128, 16), jnp.float32)
    degt = _make_deg_kernel(np_rows, deg_ch, rpt, 16)(dstp, ones16)

    xs = pl.pallas_call(
        _make_tc_prescale(n),
        out_shape=jax.ShapeDtypeStruct((NC, np_rows, d1 // NC), jnp.float32),
    )(x, W1, degt)

    agg1 = _make_agg_kernel(np_rows, nch, rpt, d1 // NC)(srcp, dstp, xs)

    ys = pl.pallas_call(
        _tc_layer2,
        out_shape=jax.ShapeDtypeStruct((NC, np_rows, d2 // NC), jnp.float32),
    )(agg1, xs, degt, W2, b1.reshape(1, d1))

    agg2 = _make_agg_kernel(np_rows, nch, rpt, d2 // NC)(srcp, dstp, ys)

    out = pl.pallas_call(
        _make_tc_head(n),
        out_shape=jax.ShapeDtypeStruct((g, nh), jnp.float32),
    )(agg2, ys, degt, batch_index.astype(jnp.int32).reshape(1, n),
      solvent_descriptors, b2.reshape(1, d2),
      lin1_W, lin1_b.reshape(1, -1),
      lin2_W, lin2_b.reshape(1, -1), lin3_W, lin3_b.reshape(1, -1),
      heads_W1, heads_b1, heads_W2, heads_b2[:, 0].reshape(nh, 1))
    return out


# speedup vs baseline: 1.0535x; 1.0535x over previous
import functools

import jax
import jax.numpy as jnp
from jax import lax
from jax.experimental import pallas as pl
from jax.experimental.pallas import tpu as pltpu
from jax.experimental.pallas import tpu_sc as plsc

NC = 2
NS = 16
LANES = 16
CHUNK = 128


def _sc_mesh():
    return plsc.VectorSubcoreMesh(core_axis_name="c", subcore_axis_name="s")


def _zero_fill(ref, rows, width):
    def body(i, carry):
        for k in range(width // LANES):
            ref[i, pl.ds(k * LANES, LANES)] = jnp.zeros((LANES,), jnp.float32)
        return carry
    lax.fori_loop(0, rows, body, 0)


def _zero_shared_rows(fill_v, nfill, acc_s, base, rpt):
    nfull, rem = rpt // nfill, rpt % nfill
    for k in range(nfull):
        pltpu.sync_copy(fill_v, acc_s.at[pl.ds(base + nfill * k, nfill)])
    if rem:
        pltpu.sync_copy(fill_v.at[pl.ds(0, rem)],
                        acc_s.at[pl.ds(base + nfill * nfull, rem)])


def _make_deg_kernel(np_rows, ch_per_tile, rpt, width=16):
    @functools.partial(
        pl.kernel,
        out_type=jax.ShapeDtypeStruct((NC, np_rows, width), jnp.float32),
        mesh=_sc_mesh(),
        compiler_params=pltpu.CompilerParams(use_tc_tiling_on_sc=False),
        scratch_types=[
            pltpu.VMEM((ch_per_tile, CHUNK), jnp.int32),
            pltpu.VMEM((128, width), jnp.float32),
            pltpu.VMEM((128, width), jnp.float32),
            pltpu.VMEM_SHARED((np_rows, width), jnp.float32),
        ],
    )
    def deg_kernel(dst_hbm, ones_hbm, out_hbm, idx_v, fill_v, ones_v, acc_s):
        c = lax.axis_index("c")
        s = lax.axis_index("s")
        wid = s * NC + c
        base = s * rpt
        _zero_fill(fill_v, 128, width)
        _zero_shared_rows(fill_v, 128, acc_s, base, rpt)
        pltpu.sync_copy(ones_hbm, ones_v)
        plsc.subcore_barrier()
        pltpu.sync_copy(dst_hbm.at[wid], idx_v)

        def step(j, carry):
            pltpu.sync_copy(ones_v, acc_s.at[idx_v.at[j]], add=True)
            return carry
        lax.fori_loop(0, ch_per_tile, step, 0)
        plsc.subcore_barrier()
        pltpu.sync_copy(acc_s.at[pl.ds(base, rpt)],
                        out_hbm.at[c].at[pl.ds(base, rpt)])

    return deg_kernel


def _make_agg_kernel(np_rows, nch, rpt, width):
    grp = 2
    assert nch % (2 * grp) == grp, "chunk count sized so the tail is one group"
    assert nch >= 3 * grp

    @functools.partial(
        pl.kernel,
        out_type=jax.ShapeDtypeStruct((NC, np_rows, width), jnp.float32),
        mesh=_sc_mesh(),
        compiler_params=pltpu.CompilerParams(use_tc_tiling_on_sc=False),
        scratch_types=[
            pltpu.VMEM((nch, CHUNK), jnp.int32),
            pltpu.VMEM((nch, CHUNK), jnp.int32),
            pltpu.VMEM((grp * 128, width), jnp.float32),
            pltpu.VMEM((grp * 128, width), jnp.float32),
            pltpu.VMEM_SHARED((np_rows, width), jnp.float32),
        ] + [pltpu.SemaphoreType.DMA] * 4,
    )
    def agg_kernel(src_hbm, dst_hbm, tab_hbm, out_hbm,
                   idxs_v, idxd_v, big_a, big_b, acc_s, ga, gb, sa, sb):
        c = lax.axis_index("c")
        s = lax.axis_index("s")
        tab = tab_hbm.at[c]
        base = s * rpt
        _zero_fill(big_a, grp * 128, width)
        _zero_shared_rows(big_a, grp * 128, acc_s, base, rpt)
        plsc.subcore_barrier()
        pltpu.sync_copy(src_hbm.at[s], idxs_v)
        pltpu.sync_copy(dst_hbm.at[s], idxd_v)

        def ggrp(j, big, sem):
            for o in range(grp):
                pltpu.async_copy(tab.at[idxs_v.at[j + o]],
                                 big.at[pl.ds(o * 128, 128)], sem)

        def gwaitgrp(big, sem):
            pltpu.make_async_copy(tab.at[pl.ds(0, grp * 128)], big,
                                  sem).wait()

        def scatgrp(j, big, sem):
            for o in range(grp):
                pltpu.async_copy(big.at[pl.ds(o * 128, 128)],
                                 acc_s.at[idxd_v.at[j + o]], sem, add=True)

        def swaitgrp(big, sem):
            pltpu.make_async_copy(big, acc_s.at[pl.ds(0, grp * 128)],
                                  sem).wait()

        ggrp(0, big_a, ga)

        def body(k, carry):
            j = 2 * grp * k
            ggrp(j + grp, big_b, gb)
            gwaitgrp(big_a, ga)
            scatgrp(j, big_a, sa)
            swaitgrp(big_a, sa)
            ggrp(j + 2 * grp, big_a, ga)
            gwaitgrp(big_b, gb)
            scatgrp(j + grp, big_b, sb)
            swaitgrp(big_b, sb)
            return carry
        lax.fori_loop(0, (nch - grp) // (2 * grp), body, 0)
        gwaitgrp(big_a, ga)
        scatgrp(nch - grp, big_a, sa)
        swaitgrp(big_a, sa)
        plsc.subcore_barrier()
        pltpu.sync_copy(acc_s.at[pl.ds(base, rpt)],
                        out_hbm.at[c].at[pl.ds(base, rpt)])

    return agg_kernel


def _make_tc_prep(q0, q, n, nch, deg_ch):
    def body(ei0_ref, ei1_ref, src_ref, dst_ref, dstd_ref):
        tail = lax.broadcasted_iota(jnp.int32, (q - q0, CHUNK), 0) * CHUNK
        tail = tail + lax.broadcasted_iota(jnp.int32, (q - q0, CHUNK), 1)
        tail = jnp.where(tail < n, tail, n)
        src = jnp.concatenate([ei0_ref[...], tail], axis=0)
        dst = jnp.concatenate([ei1_ref[...], tail], axis=0)
        src_ref[...] = src.reshape(NS, nch, CHUNK)
        dst_ref[...] = dst.reshape(NS, nch, CHUNK)
        dstd_ref[...] = dst.reshape(NC * NS, deg_ch, CHUNK)
    return body


def _dinv_from_deg(degt_ref):
    deg = degt_ref[0, :, 0:1] + degt_ref[1, :, 0:1]
    return lax.rsqrt(jnp.maximum(deg, 1e-12))


def _make_tc_prescale(n):
    def body(x_ref, w1_ref, degt_ref, xs_ref):
        dinv = _dinv_from_deg(degt_ref)
        xw = jnp.dot(x_ref[...], w1_ref[...],
                     preferred_element_type=jnp.float32)
        xw = xw * dinv[:n]
        h = xs_ref.shape[2]
        np_rows = xs_ref.shape[1]
        for cc in range(NC):
            xs_ref[cc, :n, :] = xw[:, cc * h:(cc + 1) * h]
            xs_ref[cc, n:, :] = jnp.zeros((np_rows - n, h), jnp.float32)
    return body


def _tc_layer2(agg_ref, degt_ref, w2_ref, b1_ref, ys_ref):
    dinv = _dinv_from_deg(degt_ref)
    h = agg_ref.shape[2]
    h1a = jnp.maximum(dinv * agg_ref[0] + b1_ref[:, :h], 0.0)
    h1b = jnp.maximum(dinv * agg_ref[1] + b1_ref[:, h:], 0.0)
    ys = (jnp.dot(h1a, w2_ref[:h], preferred_element_type=jnp.float32)
          + jnp.dot(h1b, w2_ref[h:],
                    preferred_element_type=jnp.float32)) * dinv
    h2 = ys_ref.shape[2]
    ys_ref[0] = ys[:, :h2]
    ys_ref[1] = ys[:, h2:]


def _make_tc_head(n):
    def body(agg2_ref, degt_ref, batch_ref, solv_ref, b2_ref,
             l1w_ref, l1bias_ref, l2w_ref, l2bias_ref,
             l3w_ref, l3bias_ref, hw1_ref, hb1_ref, hw2_ref, hb2_ref,
             out_ref):
        num_graphs = out_ref.shape[0]
        dinv = _dinv_from_deg(degt_ref)[:n]
        h2 = agg2_ref.shape[2]
        nodes_a = dinv * agg2_ref[0, :n, :]
        nodes_b = dinv * agg2_ref[1, :n, :]
        gids = lax.broadcasted_iota(jnp.int32, (num_graphs, n), 0)
        mask = (batch_ref[...] == gids).astype(jnp.float32)
        cnt = jnp.sum(mask, axis=1, keepdims=True)
        pooled_a = (jnp.dot(mask, nodes_a, preferred_element_type=jnp.float32)
                    + cnt * b2_ref[:, :h2])
        pooled_b = (jnp.dot(mask, nodes_b, preferred_element_type=jnp.float32)
                    + cnt * b2_ref[:, h2:])
        z = (jnp.dot(pooled_a, l1w_ref[:h2],
                     preferred_element_type=jnp.float32)
             + jnp.dot(pooled_b, l1w_ref[h2:2 * h2],
                       preferred_element_type=jnp.float32)
             + jnp.dot(solv_ref[...], l1w_ref[2 * h2:],
                       preferred_element_type=jnp.float32))
        z = jnp.maximum(z + l1bias_ref[...], 0.0)
        z = jnp.maximum(jnp.dot(z, l2w_ref[...],
                                preferred_element_type=jnp.float32)
                        + l2bias_ref[...], 0.0)
        z = jnp.maximum(jnp.dot(z, l3w_ref[...],
                                preferred_element_type=jnp.float32)
                        + l3bias_ref[...], 0.0)
        nh = out_ref.shape[1]
        for e in range(nh):
            hh = jnp.maximum(jnp.dot(z, hw1_ref[e],
                                     preferred_element_type=jnp.float32)
                             + hb1_ref[e:e + 1, :], 0.0)
            out_ref[:, e:e + 1] = (jnp.dot(hh, hw2_ref[e],
                                           preferred_element_type=jnp.float32)
                                   + hb2_ref[e:e + 1, :])
    return body


def kernel(x, edge_index, edge_attr, batch_index, solvent_descriptors,
           mol_fingerprints, W1, b1, W2, b2, lin1_W, lin1_b, lin2_W, lin2_b,
           lin3_W, lin3_b, heads_W1, heads_b1, heads_W2, heads_b2):
    n, din = x.shape
    e = edge_index.shape[1]
    g = solvent_descriptors.shape[0]
    d1 = W1.shape[1]
    d2 = W2.shape[1]
    nh = heads_W1.shape[0]

    rpt = -(-(n + 1) // (NS * 8)) * 8
    np_rows = rpt * NS

    q0 = -(-e // CHUNK)
    q = -(-(q0 * CHUNK + n) // CHUNK)
    while q % 64 != 32:
        q += 1
    nch = q // NS
    deg_ch = q // (NC * NS)

    if e == q0 * CHUNK:
        ei0 = edge_index[0].astype(jnp.int32).reshape(q0, CHUNK)
        ei1 = edge_index[1].astype(jnp.int32).reshape(q0, CHUNK)
    else:
        epad = jnp.full((2, q0 * CHUNK - e), jnp.int32(n))
        eip = jnp.concatenate([edge_index.astype(jnp.int32), epad], axis=1)
        ei0 = eip[0].reshape(q0, CHUNK)
        ei1 = eip[1].reshape(q0, CHUNK)

    srcp, dstp, dstd = pl.pallas_call(
        _make_tc_prep(q0, q, n, nch, deg_ch),
        out_shape=[jax.ShapeDtypeStruct((NS, nch, CHUNK), jnp.int32),
                   jax.ShapeDtypeStruct((NS, nch, CHUNK), jnp.int32),
                   jax.ShapeDtypeStruct((NC * NS, deg_ch, CHUNK), jnp.int32)],
    )(ei0, ei1)

    ones16 = jnp.ones((128, 16), jnp.float32)
    degt = _make_deg_kernel(np_rows, deg_ch, rpt, 16)(dstd, ones16)

    xs = pl.pallas_call(
        _make_tc_prescale(n),
        out_shape=jax.ShapeDtypeStruct((NC, np_rows, d1 // NC), jnp.float32),
    )(x, W1, degt)

    agg1 = _make_agg_kernel(np_rows, nch, rpt, d1 // NC)(srcp, dstp, xs)

    ys = pl.pallas_call(
        _tc_layer2,
        out_shape=jax.ShapeDtypeStruct((NC, np_rows, d2 // NC), jnp.float32),
    )(agg1, degt, W2, b1.reshape(1, d1))

    agg2 = _make_agg_kernel(np_rows, nch, rpt, d2 // NC)(srcp, dstp, ys)

    out = pl.pallas_call(
        _make_tc_head(n),
        out_shape=jax.ShapeDtypeStruct((g, nh), jnp.float32),
    )(agg2, degt, batch_index.astype(jnp.int32).reshape(1, n),
      solvent_descriptors, b2.reshape(1, d2),
      lin1_W, lin1_b.reshape(1, -1),
      lin2_W, lin2_b.reshape(1, -1), lin3_W, lin3_b.reshape(1, -1),
      heads_W1, heads_b1, heads_W2, heads_b2[:, 0].reshape(nh, 1))
    return out


# speedup vs baseline: 1.1056x; 1.0494x over previous
import functools

import jax
import jax.numpy as jnp
from jax import lax
from jax.experimental import pallas as pl
from jax.experimental.pallas import tpu as pltpu
from jax.experimental.pallas import tpu_sc as plsc

NC = 2
NS = 16
LANES = 16
CHUNK = 128


def _sc_mesh():
    return plsc.VectorSubcoreMesh(core_axis_name="c", subcore_axis_name="s")


def _zero_fill(ref, rows, width):
    def body(i, carry):
        for k in range(width // LANES):
            ref[i, pl.ds(k * LANES, LANES)] = jnp.zeros((LANES,), jnp.float32)
        return carry
    lax.fori_loop(0, rows, body, 0)


def _zero_shared_rows(fill_v, nfill, acc_s, base, rpt):
    nfull, rem = rpt // nfill, rpt % nfill
    for k in range(nfull):
        pltpu.sync_copy(fill_v, acc_s.at[pl.ds(base + nfill * k, nfill)])
    if rem:
        pltpu.sync_copy(fill_v.at[pl.ds(0, rem)],
                        acc_s.at[pl.ds(base + nfill * nfull, rem)])


def _make_deg_kernel(np_rows, ch_per_tile, rpt, width=16):
    @functools.partial(
        pl.kernel,
        out_type=jax.ShapeDtypeStruct((NC, np_rows, width), jnp.float32),
        mesh=_sc_mesh(),
        compiler_params=pltpu.CompilerParams(use_tc_tiling_on_sc=False),
        scratch_types=[
            pltpu.VMEM((ch_per_tile, CHUNK), jnp.int32),
            pltpu.VMEM((128, width), jnp.float32),
            pltpu.VMEM((128, width), jnp.float32),
            pltpu.VMEM_SHARED((np_rows, width), jnp.float32),
        ],
    )
    def deg_kernel(dst_hbm, ones_hbm, out_hbm, idx_v, fill_v, ones_v, acc_s):
        c = lax.axis_index("c")
        s = lax.axis_index("s")
        base = s * rpt
        _zero_fill(fill_v, 128, width)
        _zero_shared_rows(fill_v, 128, acc_s, base, rpt)
        pltpu.sync_copy(ones_hbm, ones_v)
        plsc.subcore_barrier()
        pltpu.sync_copy(
            dst_hbm.at[s].at[pl.ds(c * ch_per_tile, ch_per_tile)], idx_v)

        def step(j, carry):
            pltpu.sync_copy(ones_v, acc_s.at[idx_v.at[j]], add=True)
            return carry
        lax.fori_loop(0, ch_per_tile, step, 0)
        plsc.subcore_barrier()
        pltpu.sync_copy(acc_s.at[pl.ds(base, rpt)],
                        out_hbm.at[c].at[pl.ds(base, rpt)])

    return deg_kernel


def _make_agg_kernel(np_rows, nch, rpt, width):
    grp = 2
    assert nch % (2 * grp) == grp, "chunk count sized so the tail is one group"
    assert nch >= 3 * grp

    @functools.partial(
        pl.kernel,
        out_type=jax.ShapeDtypeStruct((NC, np_rows, width), jnp.float32),
        mesh=_sc_mesh(),
        compiler_params=pltpu.CompilerParams(use_tc_tiling_on_sc=False),
        scratch_types=[
            pltpu.VMEM((nch, CHUNK), jnp.int32),
            pltpu.VMEM((nch, CHUNK), jnp.int32),
            pltpu.VMEM((grp * 128, width), jnp.float32),
            pltpu.VMEM((grp * 128, width), jnp.float32),
            pltpu.VMEM_SHARED((np_rows, width), jnp.float32),
        ] + [pltpu.SemaphoreType.DMA] * 4,
    )
    def agg_kernel(src_hbm, dst_hbm, tab_hbm, out_hbm,
                   idxs_v, idxd_v, big_a, big_b, acc_s, ga, gb, sa, sb):
        c = lax.axis_index("c")
        s = lax.axis_index("s")
        tab = tab_hbm.at[c]
        base = s * rpt
        _zero_fill(big_a, grp * 128, width)
        _zero_shared_rows(big_a, grp * 128, acc_s, base, rpt)
        plsc.subcore_barrier()
        pltpu.sync_copy(src_hbm.at[s], idxs_v)
        pltpu.sync_copy(dst_hbm.at[s], idxd_v)

        def ggrp(j, big, sem):
            for o in range(grp):
                pltpu.async_copy(tab.at[idxs_v.at[j + o]],
                                 big.at[pl.ds(o * 128, 128)], sem)

        def gwaitgrp(big, sem):
            pltpu.make_async_copy(tab.at[pl.ds(0, grp * 128)], big,
                                  sem).wait()

        def scatgrp(j, big, sem):
            for o in range(grp):
                pltpu.async_copy(big.at[pl.ds(o * 128, 128)],
                                 acc_s.at[idxd_v.at[j + o]], sem, add=True)

        def swaitgrp(big, sem):
            pltpu.make_async_copy(big, acc_s.at[pl.ds(0, grp * 128)],
                                  sem).wait()

        ggrp(0, big_a, ga)

        def body(k, carry):
            j = 2 * grp * k
            ggrp(j + grp, big_b, gb)
            gwaitgrp(big_a, ga)
            scatgrp(j, big_a, sa)
            swaitgrp(big_a, sa)
            ggrp(j + 2 * grp, big_a, ga)
            gwaitgrp(big_b, gb)
            scatgrp(j + grp, big_b, sb)
            swaitgrp(big_b, sb)
            return carry
        lax.fori_loop(0, (nch - grp) // (2 * grp), body, 0)
        gwaitgrp(big_a, ga)
        scatgrp(nch - grp, big_a, sa)
        swaitgrp(big_a, sa)
        plsc.subcore_barrier()
        pltpu.sync_copy(acc_s.at[pl.ds(base, rpt)],
                        out_hbm.at[c].at[pl.ds(base, rpt)])

    return agg_kernel


def _make_tc_prep(q0, q, n, nch, deg_ch):
    del deg_ch

    def body(ei_ref, src_ref, dst_ref):
        tail = lax.broadcasted_iota(jnp.int32, (q - q0, CHUNK), 0) * CHUNK
        tail = tail + lax.broadcasted_iota(jnp.int32, (q - q0, CHUNK), 1)
        tail = jnp.where(tail < n, tail, n)
        src = jnp.concatenate([ei_ref[:q0], tail], axis=0)
        dst = jnp.concatenate([ei_ref[q0:], tail], axis=0)
        src_ref[...] = src.reshape(NS, nch, CHUNK)
        dst_ref[...] = dst.reshape(NS, nch, CHUNK)
    return body


def _dinv_from_deg(degt_ref):
    deg = degt_ref[0, :, 0:1] + degt_ref[1, :, 0:1]
    return lax.rsqrt(jnp.maximum(deg, 1e-12))


def _make_tc_prescale(n):
    def body(x_ref, w1_ref, degt_ref, xs_ref):
        dinv = _dinv_from_deg(degt_ref)
        xw = jnp.dot(x_ref[...], w1_ref[...],
                     preferred_element_type=jnp.float32)
        xw = xw * dinv[:n]
        h = xs_ref.shape[2]
        np_rows = xs_ref.shape[1]
        for cc in range(NC):
            xs_ref[cc, :n, :] = xw[:, cc * h:(cc + 1) * h]
            xs_ref[cc, n:, :] = jnp.zeros((np_rows - n, h), jnp.float32)
    return body


def _tc_layer2(agg_ref, degt_ref, w2_ref, b1_ref, ys_ref):
    dinv = _dinv_from_deg(degt_ref)
    h = agg_ref.shape[2]
    h1a = jnp.maximum(dinv * agg_ref[0] + b1_ref[:, :h], 0.0)
    h1b = jnp.maximum(dinv * agg_ref[1] + b1_ref[:, h:], 0.0)
    ys = (jnp.dot(h1a, w2_ref[:h], preferred_element_type=jnp.float32)
          + jnp.dot(h1b, w2_ref[h:],
                    preferred_element_type=jnp.float32)) * dinv
    h2 = ys_ref.shape[2]
    ys_ref[0] = ys[:, :h2]
    ys_ref[1] = ys[:, h2:]


def _make_tc_head(n):
    def body(agg2_ref, degt_ref, batch_ref, solv_ref, b2_ref,
             l1w_ref, l1bias_ref, l2w_ref, l2bias_ref,
             l3w_ref, l3bias_ref, hw1_ref, hb1_ref, hw2_ref, hb2_ref,
             out_ref):
        num_graphs = out_ref.shape[0]
        dinv = _dinv_from_deg(degt_ref)[:n]
        h2 = agg2_ref.shape[2]
        nodes_a = dinv * agg2_ref[0, :n, :]
        nodes_b = dinv * agg2_ref[1, :n, :]
        gids = lax.broadcasted_iota(jnp.int32, (num_graphs, n), 0)
        mask = (batch_ref[...] == gids).astype(jnp.float32)
        cnt = jnp.sum(mask, axis=1, keepdims=True)
        pooled_a = (jnp.dot(mask, nodes_a, preferred_element_type=jnp.float32)
                    + cnt * b2_ref[:, :h2])
        pooled_b = (jnp.dot(mask, nodes_b, preferred_element_type=jnp.float32)
                    + cnt * b2_ref[:, h2:])
        z = (jnp.dot(pooled_a, l1w_ref[:h2],
                     preferred_element_type=jnp.float32)
             + jnp.dot(pooled_b, l1w_ref[h2:2 * h2],
                       preferred_element_type=jnp.float32)
             + jnp.dot(solv_ref[...], l1w_ref[2 * h2:],
                       preferred_element_type=jnp.float32))
        z = jnp.maximum(z + l1bias_ref[...], 0.0)
        z = jnp.maximum(jnp.dot(z, l2w_ref[...],
                                preferred_element_type=jnp.float32)
                        + l2bias_ref[...], 0.0)
        z = jnp.maximum(jnp.dot(z, l3w_ref[...],
                                preferred_element_type=jnp.float32)
                        + l3bias_ref[...], 0.0)
        nh = out_ref.shape[1]
        for e in range(nh):
            hh = jnp.maximum(jnp.dot(z, hw1_ref[e],
                                     preferred_element_type=jnp.float32)
                             + hb1_ref[e:e + 1, :], 0.0)
            out_ref[:, e:e + 1] = (jnp.dot(hh, hw2_ref[e],
                                           preferred_element_type=jnp.float32)
                                   + hb2_ref[e:e + 1, :])
    return body


def kernel(x, edge_index, edge_attr, batch_index, solvent_descriptors,
           mol_fingerprints, W1, b1, W2, b2, lin1_W, lin1_b, lin2_W, lin2_b,
           lin3_W, lin3_b, heads_W1, heads_b1, heads_W2, heads_b2):
    n, din = x.shape
    e = edge_index.shape[1]
    g = solvent_descriptors.shape[0]
    d1 = W1.shape[1]
    d2 = W2.shape[1]
    nh = heads_W1.shape[0]

    rpt = -(-(n + 1) // (NS * 8)) * 8
    np_rows = rpt * NS

    q0 = -(-e // CHUNK)
    q = -(-(q0 * CHUNK + n) // CHUNK)
    while q % 64 != 32:
        q += 1
    nch = q // NS
    deg_ch = q // (NC * NS)

    if e == q0 * CHUNK:
        eif = edge_index.astype(jnp.int32).reshape(2 * q0, CHUNK)
    else:
        epad = jnp.full((2, q0 * CHUNK - e), jnp.int32(n))
        eip = jnp.concatenate([edge_index.astype(jnp.int32), epad], axis=1)
        eif = eip.reshape(2 * q0, CHUNK)

    srcp, dstp = pl.pallas_call(
        _make_tc_prep(q0, q, n, nch, deg_ch),
        out_shape=[jax.ShapeDtypeStruct((NS, nch, CHUNK), jnp.int32),
                   jax.ShapeDtypeStruct((NS, nch, CHUNK), jnp.int32)],
    )(eif)

    ones16 = jnp.ones((128, 16), jnp.float32)
    degt = _make_deg_kernel(np_rows, deg_ch, rpt, 16)(dstp, ones16)

    xs = pl.pallas_call(
        _make_tc_prescale(n),
        out_shape=jax.ShapeDtypeStruct((NC, np_rows, d1 // NC), jnp.float32),
    )(x, W1, degt)

    agg1 = _make_agg_kernel(np_rows, nch, rpt, d1 // NC)(srcp, dstp, xs)

    ys = pl.pallas_call(
        _tc_layer2,
        out_shape=jax.ShapeDtypeStruct((NC, np_rows, d2 // NC), jnp.float32),
    )(agg1, degt, W2, b1.reshape(1, d1))

    agg2 = _make_agg_kernel(np_rows, nch, rpt, d2 // NC)(srcp, dstp, ys)

    out = pl.pallas_call(
        _make_tc_head(n),
        out_shape=jax.ShapeDtypeStruct((g, nh), jnp.float32),
    )(agg2, degt, batch_index.astype(jnp.int32).reshape(1, n),
      solvent_descriptors, b2.reshape(1, d2),
      lin1_W, lin1_b.reshape(1, -1),
      lin2_W, lin2_b.reshape(1, -1), lin3_W, lin3_b.reshape(1, -1),
      heads_W1, heads_b1, heads_W2, heads_b2[:, 0].reshape(nh, 1))
    return out
